# baseline (speedup 1.0000x reference)
import functools

import jax
import jax.numpy as jnp
from jax import lax
from jax.experimental import pallas as pl
from jax.experimental.pallas import tpu as pltpu
from jax.experimental.pallas import tpu_sc as plsc

N = 10000
E = 320000
R = 8
NB = 2
EMB = 128
H = 128
P = 768

NC = 2
NS = 16
NW = NC * NS
EPW = E // NW
K = 125
NCH = EPW // K
EPT_H = E // NS
KH = 128
NCH_H = 157
EPT_HP = NCH_H * KH
NCH_W = 79
EPW_PAD = NCH_W * KH
NPT = 632
NPT_LAST = N - 15 * NPT
NSEG = N * R
NSEG_PAD = NSEG + 128


def _sc_weights_body(seg_h_hbm, seg_w_hbm, w_hbm,
                     segh_v, segw_v, ones_v, zeros_v, cnt_v, w_v, cnt_sh, sem):
    cid = lax.axis_index("c")
    sid = lax.axis_index("s")

    def fill(r, _):
        ones_v[pl.ds(r * 16, 16)] = jnp.full((16,), 1.0, jnp.float32)
        return 0
    lax.fori_loop(0, 8, fill, 0)

    def fillz(r, _):
        zeros_v[pl.ds(r * 16, 16)] = jnp.zeros((16,), jnp.float32)
        return 0
    lax.fori_loop(0, 313, fillz, 0)

    pltpu.sync_copy(zeros_v, cnt_sh.at[pl.ds(sid * 5008, 5008)])
    plsc.subcore_barrier()

    pltpu.sync_copy(seg_h_hbm.at[sid], segh_v)

    def hist(j, _):
        pltpu.sync_copy(ones_v, cnt_sh.at[segh_v.at[j]], add=True)
        return 0
    lax.fori_loop(0, NCH_H, hist, 0)
    plsc.subcore_barrier()

    pltpu.sync_copy(seg_w_hbm.at[cid, sid], segw_v)

    def gath(j, _):
        pltpu.async_copy(cnt_sh.at[segw_v.at[j]], cnt_v.at[pl.ds(j * KH, KH)], sem).wait()
        return 0
    lax.fori_loop(0, NCH_W, gath, 0)

    def inv(i, _):
        c = cnt_v[pl.ds(i * 16, 16)]
        w_v[pl.ds(i * 16, 16)] = 1.0 / jnp.maximum(c, 1.0)
        return 0
    lax.fori_loop(0, EPW_PAD // 16, inv, 0)

    base = (cid * NS + sid) * EPW
    pltpu.sync_copy(w_v.at[pl.ds(0, EPW)], w_hbm.at[pl.ds(base, EPW)])


def _sc_weights(seg_h, seg_w):
    mesh = plsc.VectorSubcoreMesh(core_axis_name="c", subcore_axis_name="s")
    return pl.kernel(
        _sc_weights_body,
        out_type=jax.ShapeDtypeStruct((E,), jnp.float32),
        mesh=mesh,
        scratch_types=[
            pltpu.VMEM((NCH_H, KH), jnp.int32),
            pltpu.VMEM((NCH_W, KH), jnp.int32),
            pltpu.VMEM((KH,), jnp.float32),
            pltpu.VMEM((5008,), jnp.float32),
            pltpu.VMEM((EPW_PAD,), jnp.float32),
            pltpu.VMEM((EPW_PAD,), jnp.float32),
            pltpu.VMEM_SHARED((NSEG_PAD,), jnp.float32),
            pltpu.SemaphoreType.DMA,
        ],
    )(seg_h, seg_w)


def _sc_edge_body(table_hbm, aux_hbm, wp_hbm, out_hbm,
                  aux_v, w_v, rows_a, rows_b, zero_v, agg_sh,
                  gs0, gs1, ss0, ss1, as0, as1, as2, as3,
                  ws0, ws1, ws2, ws3):
    cid = lax.axis_index("c")
    sid = lax.axis_index("s")
    wid = sid * NC + cid
    gs = (gs0, gs1)
    ss = (ss0, ss1)
    asm = (as0, as1, as2, as3)
    wsm = (ws0, ws1, ws2, ws3)
    rows = (rows_a, rows_b)

    for r in range(8):
        for j in range(H // 16):
            zero_v[r, pl.ds(j * 16, 16)] = jnp.zeros((16,), jnp.float32)
    nz = jnp.where(sid == NS - 1, NPT_LAST // 8, NPT // 8)

    def zc(t, _):
        pltpu.sync_copy(zero_v, agg_sh.at[pl.ds(sid * NPT + t * 8, 8)])
        return 0
    lax.fori_loop(0, nz, zc, 0)
    plsc.subcore_barrier()

    def chunk_step(i, p):
        rs = p % 2
        r_cur, r_oth = rows[rs], rows[1 - rs]
        pltpu.make_async_copy(table_hbm.at[aux_v.at[p, 0]], r_cur, gs[rs]).wait()

        @pl.when(i > 0)
        def _():
            pltpu.make_async_copy(
                r_oth, agg_sh.at[aux_v.at[(p - 1) % 4, 1]], ss[1 - rs]).wait()

        @pl.when(i + 1 < NCH)
        def _():
            pn = (p + 1) % 4
            pltpu.make_async_copy(aux_hbm.at[wid, i + 1], aux_v.at[pn],
                                  asm[pn]).wait()
            pltpu.async_copy(table_hbm.at[aux_v.at[pn, 0]], r_oth, gs[1 - rs])

        @pl.when(i + 2 < NCH)
        def _():
            pn2 = (p + 2) % 4
            pltpu.async_copy(aux_hbm.at[wid, i + 2], aux_v.at[pn2], asm[pn2])
            pltpu.async_copy(wp_hbm.at[wid, i + 2], w_v.at[pn2], wsm[pn2])

        pltpu.make_async_copy(wp_hbm.at[wid, i], w_v.at[p], wsm[p]).wait()

        def scale1(k):
            wk = plsc.load_gather(
                w_v, [jnp.full((16,), p, jnp.int32),
                      jnp.full((16,), 0, jnp.int32),
                      jnp.full((16,), k, jnp.int32)])
            for j in range(H // 16):
                r_cur[k, pl.ds(j * 16, 16)] = (
                    r_cur[k, pl.ds(j * 16, 16)] * wk)

        def mul8(m, _):
            for t in range(8):
                scale1(m * 8 + t)
            return 0
        lax.fori_loop(0, K // 8, mul8, 0)
        for t in range(K % 8):
            scale1((K // 8) * 8 + t)

        pltpu.async_copy(r_cur, agg_sh.at[aux_v.at[p, 1]], ss[rs], add=True)

    pltpu.async_copy(aux_hbm.at[wid, 0], aux_v.at[0], as0)
    pltpu.async_copy(wp_hbm.at[wid, 0], w_v.at[0], ws0)
    pltpu.make_async_copy(aux_hbm.at[wid, 0], aux_v.at[0], as0).wait()
    pltpu.async_copy(table_hbm.at[aux_v.at[0, 0]], rows_a, gs0)
    pltpu.async_copy(aux_hbm.at[wid, 1], aux_v.at[1], as1)
    pltpu.async_copy(wp_hbm.at[wid, 1], w_v.at[1], ws1)

    def group(g, _):
        for p in range(4):
            chunk_step(g * 4 + p, p)
        return 0
    lax.fori_loop(0, NCH // 4, group, 0)
    for q in range(NCH % 4):
        chunk_step(NCH - NCH % 4 + q, q)
    pltpu.make_async_copy(rows[(NCH - 1) % 2],
                          agg_sh.at[aux_v.at[(NCH - 1) % 4, 1]],
                          ss[(NCH - 1) % 2]).wait()

    plsc.subcore_barrier()

    @pl.when(sid < NS - 1)
    def _():
        pltpu.sync_copy(agg_sh.at[pl.ds(sid * NPT, NPT)],
                        out_hbm.at[cid, pl.ds(sid * NPT, NPT)])

    @pl.when(sid == NS - 1)
    def _():
        pltpu.sync_copy(agg_sh.at[pl.ds((NS - 1) * NPT, NPT_LAST)],
                        out_hbm.at[cid, pl.ds((NS - 1) * NPT, NPT_LAST)])


def _sc_edge(table, aux, wpad):
    mesh = plsc.VectorSubcoreMesh(core_axis_name="c", subcore_axis_name="s")
    return pl.kernel(
        _sc_edge_body,
        out_type=jax.ShapeDtypeStruct((NC, N, H), jnp.float32),
        mesh=mesh,
        compiler_params=pltpu.CompilerParams(needs_layout_passes=False),
        scratch_types=[
            pltpu.VMEM((4, 2, K), jnp.int32),
            pltpu.VMEM((4, 1, 128), jnp.float32),
            pltpu.VMEM((K, H), jnp.float32),
            pltpu.VMEM((K, H), jnp.float32),
            pltpu.VMEM((8, H), jnp.float32),
            pltpu.VMEM_SHARED((N, H), jnp.float32),
        ] + [pltpu.SemaphoreType.DMA] * 12,
    )(table, aux, wpad)


BLK = 1000


def _tc_proj_body(emb_ref, pw_ref, pb_ref, basis_ref, comp_ref, root_ref,
                  bias_ref, table_ref, s_ref):
    e = emb_ref[...]
    nrm = jnp.sqrt(jnp.sum(e * e, axis=1, keepdims=True))
    x = e / jnp.maximum(nrm, 1e-12)
    x = jnp.dot(x, pw_ref[...], preferred_element_type=jnp.float32) + pb_ref[...]
    t0 = jnp.dot(x, basis_ref[0], preferred_element_type=jnp.float32)
    t1 = jnp.dot(x, basis_ref[1], preferred_element_type=jnp.float32)
    comp = comp_ref[...]
    cols = [comp[r, 0] * t0 + comp[r, 1] * t1 for r in range(R)]
    table_ref[...] = jnp.concatenate(cols, axis=1)
    s_ref[...] = jnp.dot(x, root_ref[...], preferred_element_type=jnp.float32) + bias_ref[...]


def _tc_proj(emb, proj_W, proj_b, basis, comp, root, bias):
    return pl.pallas_call(
        _tc_proj_body,
        grid=(N // BLK,),
        in_specs=[
            pl.BlockSpec((BLK, EMB), lambda i: (i, 0)),
            pl.BlockSpec((EMB, P), lambda i: (0, 0)),
            pl.BlockSpec((P,), lambda i: (0,)),
            pl.BlockSpec((NB, P, H), lambda i: (0, 0, 0)),
            pl.BlockSpec((R, NB), lambda i: (0, 0)),
            pl.BlockSpec((P, H), lambda i: (0, 0)),
            pl.BlockSpec((H,), lambda i: (0,)),
        ],
        out_specs=[
            pl.BlockSpec((BLK, R * H), lambda i: (i, 0)),
            pl.BlockSpec((BLK, H), lambda i: (i, 0)),
        ],
        out_shape=[
            jax.ShapeDtypeStruct((N, R * H), jnp.float32),
            jax.ShapeDtypeStruct((N, H), jnp.float32),
        ],
    )(emb, proj_W, proj_b, basis, comp, root, bias)


def _tc_tab_body(x_ref, basis_ref, comp_ref, root_ref, bias_ref,
                 table_ref, s_ref):
    x = x_ref[...]
    t0 = jnp.dot(x, basis_ref[0], preferred_element_type=jnp.float32)
    t1 = jnp.dot(x, basis_ref[1], preferred_element_type=jnp.float32)
    comp = comp_ref[...]
    cols = [comp[r, 0] * t0 + comp[r, 1] * t1 for r in range(R)]
    table_ref[...] = jnp.concatenate(cols, axis=1)
    s_ref[...] = jnp.dot(x, root_ref[...], preferred_element_type=jnp.float32) + bias_ref[...]


def _tc_tab(x, basis, comp, root, bias):
    return pl.pallas_call(
        _tc_tab_body,
        grid=(N // BLK,),
        in_specs=[
            pl.BlockSpec((BLK, H), lambda i: (i, 0)),
            pl.BlockSpec((NB, H, H), lambda i: (0, 0, 0)),
            pl.BlockSpec((R, NB), lambda i: (0, 0)),
            pl.BlockSpec((H, H), lambda i: (0, 0)),
            pl.BlockSpec((H,), lambda i: (0,)),
        ],
        out_specs=[
            pl.BlockSpec((BLK, R * H), lambda i: (i, 0)),
            pl.BlockSpec((BLK, H), lambda i: (i, 0)),
        ],
        out_shape=[
            jax.ShapeDtypeStruct((N, R * H), jnp.float32),
            jax.ShapeDtypeStruct((N, H), jnp.float32),
        ],
    )(x, basis, comp, root, bias)


def _tc_bn_tab_body(parts_ref, s_ref, g_ref, b_ref, basis_ref, comp_ref,
                    root_ref, bias_ref, table_ref, sout_ref, ybuf, stats):
    ph = pl.program_id(0)
    blk = pl.program_id(1)

    @pl.when(ph == 0)
    def _():
        y = parts_ref[0] + parts_ref[1] + s_ref[...]
        ybuf[pl.ds(blk * BLK, BLK), :] = y
        cs = jnp.sum(y, axis=0, keepdims=True)
        cq = jnp.sum(y * y, axis=0, keepdims=True)
        prev_s = stats[0:1, :]
        prev_q = stats[1:2, :]
        stats[0:1, :] = jnp.where(blk == 0, cs, prev_s + cs)
        stats[1:2, :] = jnp.where(blk == 0, cq, prev_q + cq)

    @pl.when(ph == 1)
    def _():
        m = stats[0:1, :] * (1.0 / N)
        v = stats[1:2, :] * (1.0 / N) - m * m
        y = ybuf[pl.ds(blk * BLK, BLK), :]
        z = (y - m) * jax.lax.rsqrt(v + 1e-5) * g_ref[...] + b_ref[...]
        z = jnp.maximum(z, 0.0)
        t0 = jnp.dot(z, basis_ref[0], preferred_element_type=jnp.float32)
        t1 = jnp.dot(z, basis_ref[1], preferred_element_type=jnp.float32)
        comp = comp_ref[...]
        cols = [comp[r, 0] * t0 + comp[r, 1] * t1 for r in range(R)]
        table_ref[...] = jnp.concatenate(cols, axis=1)
        sout_ref[...] = jnp.dot(z, root_ref[...],
                                preferred_element_type=jnp.float32) + bias_ref[...]


def _tc_bn_tab(parts, s, g, b, basis, comp, root, bias):
    return pl.pallas_call(
        _tc_bn_tab_body,
        grid=(2, N // BLK),
        in_specs=[
            pl.BlockSpec((NC, BLK, H), lambda p, i: (0, i, 0)),
            pl.BlockSpec((BLK, H), lambda p, i: (i, 0)),
            pl.BlockSpec((H,), lambda p, i: (0,)),
            pl.BlockSpec((H,), lambda p, i: (0,)),
            pl.BlockSpec((NB, H, H), lambda p, i: (0, 0, 0)),
            pl.BlockSpec((R, NB), lambda p, i: (0, 0)),
            pl.BlockSpec((H, H), lambda p, i: (0, 0)),
            pl.BlockSpec((H,), lambda p, i: (0,)),
        ],
        out_specs=[
            pl.BlockSpec((BLK, R * H), lambda p, i: (i, 0)),
            pl.BlockSpec((BLK, H), lambda p, i: (i, 0)),
        ],
        out_shape=[
            jax.ShapeDtypeStruct((N, R * H), jnp.float32),
            jax.ShapeDtypeStruct((N, H), jnp.float32),
        ],
        scratch_shapes=[
            pltpu.VMEM((N, H), jnp.float32),
            pltpu.VMEM((2, H), jnp.float32),
        ],
    )(parts, s, g, b, basis, comp, root, bias)


def _tc_bn_body(parts_ref, s_ref, g_ref, b_ref, out_ref):
    y = parts_ref[0] + parts_ref[1] + s_ref[...]
    m = jnp.sum(y, axis=0, keepdims=True) * (1.0 / N)
    d = y - m
    v = jnp.sum(d * d, axis=0, keepdims=True) * (1.0 / N)
    z = d * jax.lax.rsqrt(v + 1e-5) * g_ref[...] + b_ref[...]
    out_ref[...] = jnp.maximum(z, 0.0)


def _tc_bn(parts, s, g, b):
    return pl.pallas_call(
        _tc_bn_body,
        out_shape=jax.ShapeDtypeStruct((N, H), jnp.float32),
    )(parts, s, g, b)


def kernel(edge_index, edge_type, emb, proj_W, proj_b,
           basis0, comp0, root0, bias0, g0, b0,
           basis1, comp1, root1, bias1, g1, b1,
           basis2, comp2, root2, bias2, g2, b2):
    src = edge_index[0].astype(jnp.int32)
    dst = edge_index[1].astype(jnp.int32)
    et = edge_type.astype(jnp.int32)
    idx = src * R + et
    seg = dst * R + et

    seg2 = seg.reshape(NS, EPT_H)
    padh = jnp.full((NS, EPT_HP - EPT_H), NSEG, jnp.int32)
    seg_h = jnp.concatenate([seg2, padh], axis=1).reshape(NS, NCH_H, KH)
    seg3 = seg.reshape(NC, NS, EPW)
    padw = jnp.full((NC, NS, EPW_PAD - EPW), NSEG, jnp.int32)
    seg_w = jnp.concatenate([seg3, padw], axis=2).reshape(NC, NS, NCH_W, KH)
    w = _sc_weights(seg_h, seg_w)

    aux = jnp.stack([idx.reshape(NW, NCH, K), dst.reshape(NW, NCH, K)],
                    axis=2)
    wpad = jnp.pad(w.reshape(NW, NCH, 1, K),
                   ((0, 0), (0, 0), (0, 0), (0, 128 - K)))

    table0, s0 = _tc_proj(emb, proj_W, proj_b, basis0, comp0, root0, bias0)
    parts0 = _sc_edge(table0.reshape(N * R, H), aux, wpad)
    table1, s1 = _tc_bn_tab(parts0, s0, g0, b0, basis1, comp1, root1, bias1)
    parts1 = _sc_edge(table1.reshape(N * R, H), aux, wpad)
    table2, s2 = _tc_bn_tab(parts1, s1, g1, b1, basis2, comp2, root2, bias2)
    parts2 = _sc_edge(table2.reshape(N * R, H), aux, wpad)
    x3 = _tc_bn(parts2, s2, g2, b2)
    return x3


# speedup vs baseline: 2.0311x; 2.0311x over previous
import jax
import jax.numpy as jnp
from jax import lax
from jax.experimental import pallas as pl
from jax.experimental.pallas import tpu as pltpu
from jax.experimental.pallas import tpu_sc as plsc

N = 10000
E = 320000
R = 8
NB = 2
EMB = 128
H = 128
P = 768

NC = 2
NS = 16
NW = NC * NS
EPW = E // NW
K = 125
NCH = EPW // K
EPT_H = E // NS
KH = 128
NCH_H = 157
EPT_HP = NCH_H * KH
NCH_W = 79
EPW_PAD = NCH_W * KH
NPT = 632
NPT_LAST = N - 15 * NPT
NSEG = N * R
NSEG_PAD = NSEG + 128


def _sc_weights_body(seg_h_hbm, seg_w_hbm, w_hbm,
                     segh_v, segw_v, ones_v, zeros_v, cnt_v, w_v, cnt_sh, sem):
    cid = lax.axis_index("c")
    sid = lax.axis_index("s")

    def fill(r, _):
        ones_v[pl.ds(r * 16, 16)] = jnp.full((16,), 1.0, jnp.float32)
        return 0
    lax.fori_loop(0, 8, fill, 0)

    def fillz(r, _):
        zeros_v[pl.ds(r * 16, 16)] = jnp.zeros((16,), jnp.float32)
        return 0
    lax.fori_loop(0, 313, fillz, 0)

    pltpu.sync_copy(zeros_v, cnt_sh.at[pl.ds(sid * 5008, 5008)])
    plsc.subcore_barrier()

    pltpu.sync_copy(seg_h_hbm.at[sid], segh_v)

    def hist(j, _):
        pltpu.sync_copy(ones_v, cnt_sh.at[segh_v.at[j]], add=True)
        return 0
    lax.fori_loop(0, NCH_H, hist, 0)
    plsc.subcore_barrier()

    pltpu.sync_copy(seg_w_hbm.at[cid, sid], segw_v)

    def gath(j, _):
        pltpu.async_copy(cnt_sh.at[segw_v.at[j]], cnt_v.at[pl.ds(j * KH, KH)], sem).wait()
        return 0
    lax.fori_loop(0, NCH_W, gath, 0)

    def inv(i, _):
        c = cnt_v[pl.ds(i * 16, 16)]
        w_v[pl.ds(i * 16, 16)] = 1.0 / jnp.maximum(c, 1.0)
        return 0
    lax.fori_loop(0, EPW_PAD // 16, inv, 0)

    base = (cid * NS + sid) * EPW
    pltpu.sync_copy(w_v.at[pl.ds(0, EPW)], w_hbm.at[pl.ds(base, EPW)])


def _sc_weights(seg_h, seg_w):
    mesh = plsc.VectorSubcoreMesh(core_axis_name="c", subcore_axis_name="s")
    return pl.kernel(
        _sc_weights_body,
        out_type=jax.ShapeDtypeStruct((E,), jnp.float32),
        mesh=mesh,
        scratch_types=[
            pltpu.VMEM((NCH_H, KH), jnp.int32),
            pltpu.VMEM((NCH_W, KH), jnp.int32),
            pltpu.VMEM((KH,), jnp.float32),
            pltpu.VMEM((5008,), jnp.float32),
            pltpu.VMEM((EPW_PAD,), jnp.float32),
            pltpu.VMEM((EPW_PAD,), jnp.float32),
            pltpu.VMEM_SHARED((NSEG_PAD,), jnp.float32),
            pltpu.SemaphoreType.DMA,
        ],
    )(seg_h, seg_w)


def _sc_edge_body(table_hbm, aux_hbm, wp_hbm, out_hbm,
                  aux_v, w_v, rows_a, rows_b, zero_v, agg_sh,
                  gs0, gs1, ss0, ss1, as0, as1, as2, as3,
                  ws0, ws1, ws2, ws3):
    cid = lax.axis_index("c")
    sid = lax.axis_index("s")
    wid = sid * NC + cid
    gs = (gs0, gs1)
    ss = (ss0, ss1)
    asm = (as0, as1, as2, as3)
    wsm = (ws0, ws1, ws2, ws3)
    rows = (rows_a, rows_b)

    for r in range(8):
        for j in range(H // 16):
            zero_v[r, pl.ds(j * 16, 16)] = jnp.zeros((16,), jnp.float32)
    nz = jnp.where(sid == NS - 1, NPT_LAST // 8, NPT // 8)

    def zc(t, _):
        pltpu.sync_copy(zero_v, agg_sh.at[pl.ds(sid * NPT + t * 8, 8)])
        return 0
    lax.fori_loop(0, nz, zc, 0)
    plsc.subcore_barrier()

    def chunk_step(i, p):
        rs = p % 2
        r_cur, r_oth = rows[rs], rows[1 - rs]
        pltpu.make_async_copy(table_hbm.at[aux_v.at[p, 0]], r_cur, gs[rs]).wait()

        @pl.when(i > 0)
        def _():
            pltpu.make_async_copy(
                r_oth, agg_sh.at[aux_v.at[(p - 1) % 4, 1]], ss[1 - rs]).wait()

        @pl.when(i + 1 < NCH)
        def _():
            pn = (p + 1) % 4
            pltpu.make_async_copy(aux_hbm.at[wid, i + 1], aux_v.at[pn],
                                  asm[pn]).wait()
            pltpu.async_copy(table_hbm.at[aux_v.at[pn, 0]], r_oth, gs[1 - rs])

        @pl.when(i + 2 < NCH)
        def _():
            pn2 = (p + 2) % 4
            pltpu.async_copy(aux_hbm.at[wid, i + 2], aux_v.at[pn2], asm[pn2])
            pltpu.async_copy(wp_hbm.at[wid, i + 2], w_v.at[pn2], wsm[pn2])

        pltpu.make_async_copy(wp_hbm.at[wid, i], w_v.at[p], wsm[p]).wait()

        def scale1(k):
            wk = plsc.load_gather(
                w_v, [jnp.full((16,), p, jnp.int32),
                      jnp.full((16,), 0, jnp.int32),
                      jnp.full((16,), k, jnp.int32)])
            for j in range(H // 16):
                r_cur[k, pl.ds(j * 16, 16)] = (
                    r_cur[k, pl.ds(j * 16, 16)] * wk)

        def mul8(m, _):
            for t in range(8):
                scale1(m * 8 + t)
            return 0
        lax.fori_loop(0, K // 8, mul8, 0)
        for t in range(K % 8):
            scale1((K // 8) * 8 + t)

        pltpu.async_copy(r_cur, agg_sh.at[aux_v.at[p, 1]], ss[rs], add=True)

    pltpu.async_copy(aux_hbm.at[wid, 0], aux_v.at[0], as0)
    pltpu.async_copy(wp_hbm.at[wid, 0], w_v.at[0], ws0)
    pltpu.make_async_copy(aux_hbm.at[wid, 0], aux_v.at[0], as0).wait()
    pltpu.async_copy(table_hbm.at[aux_v.at[0, 0]], rows_a, gs0)
    pltpu.async_copy(aux_hbm.at[wid, 1], aux_v.at[1], as1)
    pltpu.async_copy(wp_hbm.at[wid, 1], w_v.at[1], ws1)

    def group(g, _):
        for p in range(4):
            chunk_step(g * 4 + p, p)
        return 0
    lax.fori_loop(0, NCH // 4, group, 0)
    for q in range(NCH % 4):
        chunk_step(NCH - NCH % 4 + q, q)
    pltpu.make_async_copy(rows[(NCH - 1) % 2],
                          agg_sh.at[aux_v.at[(NCH - 1) % 4, 1]],
                          ss[(NCH - 1) % 2]).wait()

    plsc.subcore_barrier()

    @pl.when(sid < NS - 1)
    def _():
        pltpu.sync_copy(agg_sh.at[pl.ds(sid * NPT, NPT)],
                        out_hbm.at[cid, pl.ds(sid * NPT, NPT)])

    @pl.when(sid == NS - 1)
    def _():
        pltpu.sync_copy(agg_sh.at[pl.ds((NS - 1) * NPT, NPT_LAST)],
                        out_hbm.at[cid, pl.ds((NS - 1) * NPT, NPT_LAST)])


def _sc_edge(table, aux, wpad):
    mesh = plsc.VectorSubcoreMesh(core_axis_name="c", subcore_axis_name="s")
    return pl.kernel(
        _sc_edge_body,
        out_type=jax.ShapeDtypeStruct((NC, N, H), jnp.float32),
        mesh=mesh,
        compiler_params=pltpu.CompilerParams(needs_layout_passes=False),
        scratch_types=[
            pltpu.VMEM((4, 2, K), jnp.int32),
            pltpu.VMEM((4, 1, 128), jnp.float32),
            pltpu.VMEM((K, H), jnp.float32),
            pltpu.VMEM((K, H), jnp.float32),
            pltpu.VMEM((8, H), jnp.float32),
            pltpu.VMEM_SHARED((N, H), jnp.float32),
        ] + [pltpu.SemaphoreType.DMA] * 12,
    )(table, aux, wpad)


BLK = 1000


def _tc_proj_body(emb_ref, pw_ref, pb_ref, basis_ref, comp_ref, root_ref,
                  bias_ref, table_ref, s_ref):
    e = emb_ref[...]
    nrm = jnp.sqrt(jnp.sum(e * e, axis=1, keepdims=True))
    x = e / jnp.maximum(nrm, 1e-12)
    x = jnp.dot(x, pw_ref[...], preferred_element_type=jnp.float32) + pb_ref[...]
    t0 = jnp.dot(x, basis_ref[0], preferred_element_type=jnp.float32)
    t1 = jnp.dot(x, basis_ref[1], preferred_element_type=jnp.float32)
    comp = comp_ref[...]
    cols = [comp[r, 0] * t0 + comp[r, 1] * t1 for r in range(R)]
    table_ref[...] = jnp.concatenate(cols, axis=1)
    s_ref[...] = jnp.dot(x, root_ref[...], preferred_element_type=jnp.float32) + bias_ref[...]


def _tc_proj(emb, proj_W, proj_b, basis, comp, root, bias):
    return pl.pallas_call(
        _tc_proj_body,
        grid=(N // BLK,),
        in_specs=[
            pl.BlockSpec((BLK, EMB), lambda i: (i, 0)),
            pl.BlockSpec((EMB, P), lambda i: (0, 0)),
            pl.BlockSpec((P,), lambda i: (0,)),
            pl.BlockSpec((NB, P, H), lambda i: (0, 0, 0)),
            pl.BlockSpec((R, NB), lambda i: (0, 0)),
            pl.BlockSpec((P, H), lambda i: (0, 0)),
            pl.BlockSpec((H,), lambda i: (0,)),
        ],
        out_specs=[
            pl.BlockSpec((BLK, R * H), lambda i: (i, 0)),
            pl.BlockSpec((BLK, H), lambda i: (i, 0)),
        ],
        out_shape=[
            jax.ShapeDtypeStruct((N, R * H), jnp.float32),
            jax.ShapeDtypeStruct((N, H), jnp.float32),
        ],
    )(emb, proj_W, proj_b, basis, comp, root, bias)


def _tc_tab_body(x_ref, basis_ref, comp_ref, root_ref, bias_ref,
                 table_ref, s_ref):
    x = x_ref[...]
    t0 = jnp.dot(x, basis_ref[0], preferred_element_type=jnp.float32)
    t1 = jnp.dot(x, basis_ref[1], preferred_element_type=jnp.float32)
    comp = comp_ref[...]
    cols = [comp[r, 0] * t0 + comp[r, 1] * t1 for r in range(R)]
    table_ref[...] = jnp.concatenate(cols, axis=1)
    s_ref[...] = jnp.dot(x, root_ref[...], preferred_element_type=jnp.float32) + bias_ref[...]


def _tc_tab(x, basis, comp, root, bias):
    return pl.pallas_call(
        _tc_tab_body,
        grid=(N // BLK,),
        in_specs=[
            pl.BlockSpec((BLK, H), lambda i: (i, 0)),
            pl.BlockSpec((NB, H, H), lambda i: (0, 0, 0)),
            pl.BlockSpec((R, NB), lambda i: (0, 0)),
            pl.BlockSpec((H, H), lambda i: (0, 0)),
            pl.BlockSpec((H,), lambda i: (0,)),
        ],
        out_specs=[
            pl.BlockSpec((BLK, R * H), lambda i: (i, 0)),
            pl.BlockSpec((BLK, H), lambda i: (i, 0)),
        ],
        out_shape=[
            jax.ShapeDtypeStruct((N, R * H), jnp.float32),
            jax.ShapeDtypeStruct((N, H), jnp.float32),
        ],
    )(x, basis, comp, root, bias)


def _tc_bn_body(parts_ref, s_ref, g_ref, b_ref, out_ref):
    y = parts_ref[0] + parts_ref[1] + s_ref[...]
    m = jnp.sum(y, axis=0, keepdims=True) * (1.0 / N)
    d = y - m
    v = jnp.sum(d * d, axis=0, keepdims=True) * (1.0 / N)
    z = d * jax.lax.rsqrt(v + 1e-5) * g_ref[...] + b_ref[...]
    out_ref[...] = jnp.maximum(z, 0.0)


def _tc_bn(parts, s, g, b):
    return pl.pallas_call(
        _tc_bn_body,
        out_shape=jax.ShapeDtypeStruct((N, H), jnp.float32),
    )(parts, s, g, b)


def kernel(edge_index, edge_type, emb, proj_W, proj_b,
           basis0, comp0, root0, bias0, g0, b0,
           basis1, comp1, root1, bias1, g1, b1,
           basis2, comp2, root2, bias2, g2, b2):
    src = edge_index[0].astype(jnp.int32)
    dst = edge_index[1].astype(jnp.int32)
    et = edge_type.astype(jnp.int32)
    idx = src * R + et
    seg = dst * R + et

    seg2 = seg.reshape(NS, EPT_H)
    padh = jnp.full((NS, EPT_HP - EPT_H), NSEG, jnp.int32)
    seg_h = jnp.concatenate([seg2, padh], axis=1).reshape(NS, NCH_H, KH)
    seg3 = seg.reshape(NC, NS, EPW)
    padw = jnp.full((NC, NS, EPW_PAD - EPW), NSEG, jnp.int32)
    seg_w = jnp.concatenate([seg3, padw], axis=2).reshape(NC, NS, NCH_W, KH)

    w = _sc_weights(seg_h, seg_w)

    aux = jnp.stack([idx.reshape(NW, NCH, K), dst.reshape(NW, NCH, K)],
                    axis=2)
    wpad = jnp.pad(w.reshape(NW, NCH, 1, K),
                   ((0, 0), (0, 0), (0, 0), (0, 128 - K)))

    table0, s0 = _tc_proj(emb, proj_W, proj_b, basis0, comp0, root0, bias0)
    parts0 = _sc_edge(table0.reshape(N * R, H), aux, wpad)
    x1 = _tc_bn(parts0, s0, g0, b0)
    table1, s1 = _tc_tab(x1, basis1, comp1, root1, bias1)
    parts1 = _sc_edge(table1.reshape(N * R, H), aux, wpad)
    x2 = _tc_bn(parts1, s1, g1, b1)
    table2, s2 = _tc_tab(x2, basis2, comp2, root2, bias2)
    parts2 = _sc_edge(table2.reshape(N * R, H), aux, wpad)
    x3 = _tc_bn(parts2, s2, g2, b2)
    return x3


# speedup vs baseline: 2.0347x; 1.0018x over previous
import jax
import jax.numpy as jnp
from jax import lax
from jax.experimental import pallas as pl
from jax.experimental.pallas import tpu as pltpu
from jax.experimental.pallas import tpu_sc as plsc

N = 10000
E = 320000
R = 8
NB = 2
EMB = 128
H = 128
P = 768

NC = 2
NS = 16
NW = NC * NS
EPW = E // NW
K = 125
NCH = EPW // K
EPT_H = E // NS
KH = 128
NCH_H = 157
EPT_HP = NCH_H * KH
NCH_W = 79
EPW_PAD = NCH_W * KH
NPT = 632
NPT_LAST = N - 15 * NPT
NSEG = N * R
NSEG_PAD = NSEG + 128


def _sc_weights_body(seg_h_hbm, seg_w_hbm, w_hbm,
                     segh_v, segw_v, ones_v, zeros_v, cnt_v, w_v, cnt_sh, sem):
    cid = lax.axis_index("c")
    sid = lax.axis_index("s")

    def fill(r, _):
        ones_v[pl.ds(r * 16, 16)] = jnp.full((16,), 1.0, jnp.float32)
        return 0
    lax.fori_loop(0, 8, fill, 0)

    def fillz(r, _):
        zeros_v[pl.ds(r * 16, 16)] = jnp.zeros((16,), jnp.float32)
        return 0
    lax.fori_loop(0, 313, fillz, 0)

    pltpu.sync_copy(zeros_v, cnt_sh.at[pl.ds(sid * 5008, 5008)])
    plsc.subcore_barrier()

    pltpu.sync_copy(seg_h_hbm.at[sid], segh_v)

    def hist(j, _):
        pltpu.sync_copy(ones_v, cnt_sh.at[segh_v.at[j]], add=True)
        return 0
    lax.fori_loop(0, NCH_H, hist, 0)
    plsc.subcore_barrier()

    pltpu.sync_copy(seg_w_hbm.at[cid, sid], segw_v)

    def gath(j, _):
        pltpu.async_copy(cnt_sh.at[segw_v.at[j]], cnt_v.at[pl.ds(j * KH, KH)], sem).wait()
        return 0
    lax.fori_loop(0, NCH_W, gath, 0)

    def inv(i, _):
        c = cnt_v[pl.ds(i * 16, 16)]
        w_v[pl.ds(i * 16, 16)] = 1.0 / jnp.maximum(c, 1.0)
        return 0
    lax.fori_loop(0, EPW_PAD // 16, inv, 0)

    base = (cid * NS + sid) * EPW
    pltpu.sync_copy(w_v.at[pl.ds(0, EPW)], w_hbm.at[pl.ds(base, EPW)])


def _sc_weights(seg_h, seg_w):
    mesh = plsc.VectorSubcoreMesh(core_axis_name="c", subcore_axis_name="s")
    return pl.kernel(
        _sc_weights_body,
        out_type=jax.ShapeDtypeStruct((E,), jnp.float32),
        mesh=mesh,
        scratch_types=[
            pltpu.VMEM((NCH_H, KH), jnp.int32),
            pltpu.VMEM((NCH_W, KH), jnp.int32),
            pltpu.VMEM((KH,), jnp.float32),
            pltpu.VMEM((5008,), jnp.float32),
            pltpu.VMEM((EPW_PAD,), jnp.float32),
            pltpu.VMEM((EPW_PAD,), jnp.float32),
            pltpu.VMEM_SHARED((NSEG_PAD,), jnp.float32),
            pltpu.SemaphoreType.DMA,
        ],
    )(seg_h, seg_w)


def _sc_edge_body(table_hbm, aux_hbm, wp_hbm, out_hbm,
                  aux_v, w_v, rows_a, rows_b, zero_v, agg_sh,
                  gs0, gs1, ss0, ss1, as0, as1, as2, as3,
                  ws0, ws1, ws2, ws3):
    cid = lax.axis_index("c")
    sid = lax.axis_index("s")
    wid = sid * NC + cid
    gs = (gs0, gs1)
    ss = (ss0, ss1)
    asm = (as0, as1, as2, as3)
    wsm = (ws0, ws1, ws2, ws3)
    rows = (rows_a, rows_b)

    for r in range(8):
        for j in range(H // 16):
            zero_v[r, pl.ds(j * 16, 16)] = jnp.zeros((16,), jnp.float32)
    nz = jnp.where(sid == NS - 1, NPT_LAST // 8, NPT // 8)

    def zc(t, _):
        pltpu.sync_copy(zero_v, agg_sh.at[pl.ds(sid * NPT + t * 8, 8)])
        return 0
    lax.fori_loop(0, nz, zc, 0)
    plsc.subcore_barrier()

    def chunk_step(i, p):
        rs = p % 2
        r_cur, r_oth = rows[rs], rows[1 - rs]
        pltpu.make_async_copy(table_hbm.at[aux_v.at[p, 0]], r_cur, gs[rs]).wait()

        @pl.when(i > 0)
        def _():
            pltpu.make_async_copy(
                r_oth, agg_sh.at[aux_v.at[(p - 1) % 4, 1]], ss[1 - rs]).wait()

        @pl.when(i + 1 < NCH)
        def _():
            pn = (p + 1) % 4
            pltpu.make_async_copy(aux_hbm.at[wid, i + 1], aux_v.at[pn],
                                  asm[pn]).wait()
            pltpu.async_copy(table_hbm.at[aux_v.at[pn, 0]], r_oth, gs[1 - rs])

        @pl.when(i + 2 < NCH)
        def _():
            pn2 = (p + 2) % 4
            pltpu.async_copy(aux_hbm.at[wid, i + 2], aux_v.at[pn2], asm[pn2])
            pltpu.async_copy(wp_hbm.at[wid, i + 2], w_v.at[pn2], wsm[pn2])

        pltpu.make_async_copy(wp_hbm.at[wid, i], w_v.at[p], wsm[p]).wait()

        def scale1(k):
            wk = plsc.load_gather(
                w_v, [jnp.full((16,), p, jnp.int32),
                      jnp.full((16,), 0, jnp.int32),
                      jnp.full((16,), k, jnp.int32)])
            for j in range(H // 16):
                r_cur[k, pl.ds(j * 16, 16)] = (
                    r_cur[k, pl.ds(j * 16, 16)] * wk)

        def mul8(m, _):
            for t in range(8):
                scale1(m * 8 + t)
            return 0
        lax.fori_loop(0, K // 8, mul8, 0)
        for t in range(K % 8):
            scale1((K // 8) * 8 + t)

        pltpu.async_copy(r_cur, agg_sh.at[aux_v.at[p, 1]], ss[rs], add=True)

    pltpu.async_copy(aux_hbm.at[wid, 0], aux_v.at[0], as0)
    pltpu.async_copy(wp_hbm.at[wid, 0], w_v.at[0], ws0)
    pltpu.make_async_copy(aux_hbm.at[wid, 0], aux_v.at[0], as0).wait()
    pltpu.async_copy(table_hbm.at[aux_v.at[0, 0]], rows_a, gs0)
    pltpu.async_copy(aux_hbm.at[wid, 1], aux_v.at[1], as1)
    pltpu.async_copy(wp_hbm.at[wid, 1], w_v.at[1], ws1)

    def group(g, _):
        for p in range(4):
            chunk_step(g * 4 + p, p)
        return 0
    lax.fori_loop(0, NCH // 4, group, 0)
    for q in range(NCH % 4):
        chunk_step(NCH - NCH % 4 + q, q)
    pltpu.make_async_copy(rows[(NCH - 1) % 2],
                          agg_sh.at[aux_v.at[(NCH - 1) % 4, 1]],
                          ss[(NCH - 1) % 2]).wait()

    plsc.subcore_barrier()

    @pl.when(sid < NS - 1)
    def _():
        pltpu.sync_copy(agg_sh.at[pl.ds(sid * NPT, NPT)],
                        out_hbm.at[cid, pl.ds(sid * NPT, NPT)])

    @pl.when(sid == NS - 1)
    def _():
        pltpu.sync_copy(agg_sh.at[pl.ds((NS - 1) * NPT, NPT_LAST)],
                        out_hbm.at[cid, pl.ds((NS - 1) * NPT, NPT_LAST)])


def _sc_edge(table, aux, wpad):
    mesh = plsc.VectorSubcoreMesh(core_axis_name="c", subcore_axis_name="s")
    return pl.kernel(
        _sc_edge_body,
        out_type=jax.ShapeDtypeStruct((NC, N, H), jnp.float32),
        mesh=mesh,
        compiler_params=pltpu.CompilerParams(needs_layout_passes=False),
        scratch_types=[
            pltpu.VMEM((4, 2, K), jnp.int32),
            pltpu.VMEM((4, 1, 128), jnp.float32),
            pltpu.VMEM((K, H), jnp.float32),
            pltpu.VMEM((K, H), jnp.float32),
            pltpu.VMEM((8, H), jnp.float32),
            pltpu.VMEM_SHARED((N, H), jnp.float32),
        ] + [pltpu.SemaphoreType.DMA] * 12,
    )(table, aux, wpad)


BLK = 1000


def _tc_proj_body(emb_ref, pw_ref, pb_ref, basis_ref, comp_ref, root_ref,
                  bias_ref, table_ref, s_ref):
    e = emb_ref[...]
    nrm = jnp.sqrt(jnp.sum(e * e, axis=1, keepdims=True))
    x = e / jnp.maximum(nrm, 1e-12)
    x = jnp.dot(x, pw_ref[...], preferred_element_type=jnp.float32) + pb_ref[...]
    t0 = jnp.dot(x, basis_ref[0], preferred_element_type=jnp.float32)
    t1 = jnp.dot(x, basis_ref[1], preferred_element_type=jnp.float32)
    comp = comp_ref[...]
    cols = [comp[r, 0] * t0 + comp[r, 1] * t1 for r in range(R)]
    table_ref[...] = jnp.concatenate(cols, axis=1)
    s_ref[...] = jnp.dot(x, root_ref[...], preferred_element_type=jnp.float32) + bias_ref[...]


def _tc_proj(emb, proj_W, proj_b, basis, comp, root, bias):
    return pl.pallas_call(
        _tc_proj_body,
        grid=(N // BLK,),
        in_specs=[
            pl.BlockSpec((BLK, EMB), lambda i: (i, 0)),
            pl.BlockSpec((EMB, P), lambda i: (0, 0)),
            pl.BlockSpec((P,), lambda i: (0,)),
            pl.BlockSpec((NB, P, H), lambda i: (0, 0, 0)),
            pl.BlockSpec((R, NB), lambda i: (0, 0)),
            pl.BlockSpec((P, H), lambda i: (0, 0)),
            pl.BlockSpec((H,), lambda i: (0,)),
        ],
        out_specs=[
            pl.BlockSpec((BLK, R * H), lambda i: (i, 0)),
            pl.BlockSpec((BLK, H), lambda i: (i, 0)),
        ],
        out_shape=[
            jax.ShapeDtypeStruct((N, R * H), jnp.float32),
            jax.ShapeDtypeStruct((N, H), jnp.float32),
        ],
    )(emb, proj_W, proj_b, basis, comp, root, bias)


def _tc_tab_body(x_ref, basis_ref, comp_ref, root_ref, bias_ref,
                 table_ref, s_ref):
    x = x_ref[...]
    t0 = jnp.dot(x, basis_ref[0], preferred_element_type=jnp.float32)
    t1 = jnp.dot(x, basis_ref[1], preferred_element_type=jnp.float32)
    comp = comp_ref[...]
    cols = [comp[r, 0] * t0 + comp[r, 1] * t1 for r in range(R)]
    table_ref[...] = jnp.concatenate(cols, axis=1)
    s_ref[...] = jnp.dot(x, root_ref[...], preferred_element_type=jnp.float32) + bias_ref[...]


def _tc_tab(x, basis, comp, root, bias):
    return pl.pallas_call(
        _tc_tab_body,
        grid=(N // BLK,),
        in_specs=[
            pl.BlockSpec((BLK, H), lambda i: (i, 0)),
            pl.BlockSpec((NB, H, H), lambda i: (0, 0, 0)),
            pl.BlockSpec((R, NB), lambda i: (0, 0)),
            pl.BlockSpec((H, H), lambda i: (0, 0)),
            pl.BlockSpec((H,), lambda i: (0,)),
        ],
        out_specs=[
            pl.BlockSpec((BLK, R * H), lambda i: (i, 0)),
            pl.BlockSpec((BLK, H), lambda i: (i, 0)),
        ],
        out_shape=[
            jax.ShapeDtypeStruct((N, R * H), jnp.float32),
            jax.ShapeDtypeStruct((N, H), jnp.float32),
        ],
    )(x, basis, comp, root, bias)


def _tc_bn_body(parts_ref, s_ref, g_ref, b_ref, out_ref):
    y = parts_ref[0] + parts_ref[1] + s_ref[...]
    ones8 = jnp.ones((8, N), jnp.float32)
    m = jnp.dot(ones8, y, preferred_element_type=jnp.float32)[0:1] * (1.0 / N)
    d = y - m
    v = jnp.dot(ones8, d * d,
                preferred_element_type=jnp.float32)[0:1] * (1.0 / N)
    z = d * jax.lax.rsqrt(v + 1e-5) * g_ref[...] + b_ref[...]
    out_ref[...] = jnp.maximum(z, 0.0)


def _tc_bn(parts, s, g, b):
    return pl.pallas_call(
        _tc_bn_body,
        out_shape=jax.ShapeDtypeStruct((N, H), jnp.float32),
    )(parts, s, g, b)


def kernel(edge_index, edge_type, emb, proj_W, proj_b,
           basis0, comp0, root0, bias0, g0, b0,
           basis1, comp1, root1, bias1, g1, b1,
           basis2, comp2, root2, bias2, g2, b2):
    src = edge_index[0].astype(jnp.int32)
    dst = edge_index[1].astype(jnp.int32)
    et = edge_type.astype(jnp.int32)
    idx = src * R + et
    seg = dst * R + et

    seg2 = seg.reshape(NS, EPT_H)
    padh = jnp.full((NS, EPT_HP - EPT_H), NSEG, jnp.int32)
    seg_h = jnp.concatenate([seg2, padh], axis=1).reshape(NS, NCH_H, KH)
    seg3 = seg.reshape(NC, NS, EPW)
    padw = jnp.full((NC, NS, EPW_PAD - EPW), NSEG, jnp.int32)
    seg_w = jnp.concatenate([seg3, padw], axis=2).reshape(NC, NS, NCH_W, KH)

    w = _sc_weights(seg_h, seg_w)

    aux = jnp.stack([idx.reshape(NW, NCH, K), dst.reshape(NW, NCH, K)],
                    axis=2)
    wpad = jnp.pad(w.reshape(NW, NCH, 1, K),
                   ((0, 0), (0, 0), (0, 0), (0, 128 - K)))

    table0, s0 = _tc_proj(emb, proj_W, proj_b, basis0, comp0, root0, bias0)
    parts0 = _sc_edge(table0.reshape(N * R, H), aux, wpad)
    x1 = _tc_bn(parts0, s0, g0, b0)
    table1, s1 = _tc_tab(x1, basis1, comp1, root1, bias1)
    parts1 = _sc_edge(table1.reshape(N * R, H), aux, wpad)
    x2 = _tc_bn(parts1, s1, g1, b1)
    table2, s2 = _tc_tab(x2, basis2, comp2, root2, bias2)
    parts2 = _sc_edge(table2.reshape(N * R, H), aux, wpad)
    x3 = _tc_bn(parts2, s2, g2, b2)
    return x3
